# baseline (speedup 1.0000x reference)
import functools

import jax
import jax.numpy as jnp
from jax import lax
from jax.experimental import pallas as pl
from jax.experimental.pallas import tpu as pltpu
from jax.experimental.pallas import tpu_sc as plsc

TOK = 2048
HID = 768
NE = 8
BLK = 256
NT = 24
PAD_T = NT * BLK
NW = 32
TPW = TOK // NW


def _router_body(h_ref, gw_ref, p1_ref, p2_ref, w1_ref, w2_ref, te_ref,
                 nt_ref, aux_ref):
    h = h_ref[...]
    gw = gw_ref[...]
    logits = lax.dot_general(h, gw, (((1,), (1,)), ((), ())),
                             preferred_element_type=jnp.float32)
    iota_e = lax.broadcasted_iota(jnp.int32, (TOK, NE), 1)
    m1 = jnp.max(logits, axis=1, keepdims=True)
    i1 = jnp.min(jnp.where(logits == m1, iota_e, NE), axis=1, keepdims=True)
    masked = jnp.where(iota_e == i1, -1e30, logits)
    m2 = jnp.max(masked, axis=1, keepdims=True)
    i2 = jnp.min(jnp.where(masked == m2, iota_e, NE), axis=1, keepdims=True)
    e2 = jnp.exp(m2 - m1)
    w1_ref[...] = 1.0 / (1.0 + e2)
    w2_ref[...] = e2 / (1.0 + e2)
    p = jnp.exp(logits - m1)
    p = p / jnp.sum(p, axis=1, keepdims=True)
    avg_prob = jnp.mean(p, axis=0, keepdims=True)
    m1h = (iota_e == i1).astype(jnp.float32)
    m2h = (iota_e == i2).astype(jnp.float32)
    cnt1 = jnp.sum(m1h, axis=0, keepdims=True)
    cnt = cnt1 + jnp.sum(m2h, axis=0, keepdims=True)
    aux_ref[...] = jnp.sum(cnt * (1.0 / TOK) * avg_prob * (0.01 * NE),
                           axis=1, keepdims=True)
    rr = lax.broadcasted_iota(jnp.int32, (TOK, TOK), 0)
    cc = lax.broadcasted_iota(jnp.int32, (TOK, TOK), 1)
    tri = (cc < rr).astype(jnp.bfloat16)
    m12 = jnp.concatenate([m1h, m2h], axis=1).astype(jnp.bfloat16)
    csum = lax.dot_general(tri, m12, (((1,), (0,)), ((), ())),
                           preferred_element_type=jnp.float32)
    c1 = csum[:, :NE]
    c2 = csum[:, NE:]
    sz = jnp.floor((cnt + (BLK - 1.0)) * (1.0 / BLK)) * BLK
    re8 = lax.broadcasted_iota(jnp.int32, (NE, NE), 0)
    ce8 = lax.broadcasted_iota(jnp.int32, (NE, NE), 1)
    triu = (re8 < ce8).astype(jnp.float32)
    g = lax.dot_general(sz, triu, (((1,), (0,)), ((), ())),
                        preferred_element_type=jnp.float32)
    gnext = g + sz
    nrows = jnp.sum(sz, axis=1, keepdims=True)
    nt_ref[...] = (nrows * (1.0 / BLK)).astype(jnp.int32)
    p1 = jnp.sum(m1h * (g + c1), axis=1, keepdims=True)
    p2 = jnp.sum(m2h * (g + cnt1 + c2), axis=1, keepdims=True)
    p1_ref[...] = p1.astype(jnp.int32)
    p2_ref[...] = p2.astype(jnp.int32)
    tt = lax.broadcasted_iota(jnp.int32, (NT, NE), 0).astype(jnp.float32) \
        * float(BLK)
    texp = jnp.sum((gnext <= tt).astype(jnp.float32), axis=1, keepdims=True)
    maxte = jnp.sum((gnext <= nrows - BLK).astype(jnp.float32))
    te_ref[...] = jnp.minimum(texp, maxte).astype(jnp.int32)


def _expert_body(te_ref, nt_ref, x_ref, wg_ref, wl_ref, bg_ref, bl_ref,
                 w2_ref, b2_ref, o_ref):
    t = pl.program_id(0)

    @pl.when(t < nt_ref[0])
    def _():
        e = te_ref[t]
        x = x_ref[...].astype(jnp.bfloat16)
        hg = lax.dot_general(x, wg_ref[e], (((1,), (1,)), ((), ())),
                             preferred_element_type=jnp.float32)
        hl = lax.dot_general(x, wl_ref[e], (((1,), (1,)), ((), ())),
                             preferred_element_type=jnp.float32)
        xg = jnp.minimum(hg + bg_ref[e], 7.0)
        xl = jnp.clip(hl + bl_ref[e], -7.0, 7.0)
        act = xg * (1.0 / (1.0 + jnp.exp(-1.702 * xg))) * (xl + 1.0)
        o = lax.dot_general(act.astype(jnp.bfloat16), w2_ref[e],
                            (((1,), (1,)), ((), ())),
                            preferred_element_type=jnp.float32)
        o_ref[...] = o + b2_ref[e]


def _sc_mesh():
    return plsc.VectorSubcoreMesh(core_axis_name="c", subcore_axis_name="s")


def _dispatch(h, p1f, p2f):
    @functools.partial(
        pl.kernel,
        out_type=jax.ShapeDtypeStruct((PAD_T, HID), jnp.float32),
        mesh=_sc_mesh(),
        scratch_types=[
            pltpu.VMEM((TPW,), jnp.int32),
            pltpu.VMEM((TPW,), jnp.int32),
            pltpu.VMEM((TPW, HID), jnp.float32),
            pltpu.SemaphoreType.DMA,
            pltpu.SemaphoreType.DMA,
        ],
    )
    def disp(h_hbm, p1_hbm, p2_hbm, xg_hbm, i1_v, i2_v, rows_v, sem, sem2):
        wid = lax.axis_index("s") * 2 + lax.axis_index("c")
        base = wid * TPW
        a1 = pltpu.async_copy(p1_hbm.at[pl.ds(base, TPW)], i1_v, sem2)
        a2 = pltpu.async_copy(p2_hbm.at[pl.ds(base, TPW)], i2_v, sem2)
        a3 = pltpu.async_copy(h_hbm.at[pl.ds(base, TPW)], rows_v, sem2)
        a1.wait()
        a2.wait()
        a3.wait()
        c1 = pltpu.async_copy(rows_v, xg_hbm.at[i1_v], sem)
        c2 = pltpu.async_copy(rows_v, xg_hbm.at[i2_v], sem)
        c1.wait()
        c2.wait()

    return disp(h, p1f, p2f)


def _combine(og, p1f, p2f, w1f, w2f):
    @functools.partial(
        pl.kernel,
        out_type=jax.ShapeDtypeStruct((TOK, HID), jnp.float32),
        mesh=_sc_mesh(),
        scratch_types=[
            pltpu.VMEM((TPW,), jnp.int32),
            pltpu.VMEM((TPW,), jnp.int32),
            pltpu.VMEM((TPW + 16,), jnp.float32),
            pltpu.VMEM((TPW + 16,), jnp.float32),
            pltpu.VMEM((TPW, HID), jnp.float32),
            pltpu.VMEM((TPW, HID), jnp.float32),
            pltpu.SemaphoreType.DMA,
            pltpu.SemaphoreType.DMA,
        ],
    )
    def comb(og_hbm, p1_hbm, p2_hbm, w1_hbm, w2_hbm, out_hbm,
             i1_v, i2_v, a_v, b_v, r1_v, r2_v, sem, sem2):
        wid = lax.axis_index("s") * 2 + lax.axis_index("c")
        base = wid * TPW
        a1 = pltpu.async_copy(p1_hbm.at[pl.ds(base, TPW)], i1_v, sem2)
        a2 = pltpu.async_copy(p2_hbm.at[pl.ds(base, TPW)], i2_v, sem2)
        a3 = pltpu.async_copy(w1_hbm.at[pl.ds(base, TPW)],
                              a_v.at[pl.ds(0, TPW)], sem2)
        a4 = pltpu.async_copy(w2_hbm.at[pl.ds(base, TPW)],
                              b_v.at[pl.ds(0, TPW)], sem2)
        a1.wait()
        a2.wait()
        a3.wait()
        a4.wait()
        g1 = pltpu.async_copy(og_hbm.at[i1_v], r1_v, sem)
        g2 = pltpu.async_copy(og_hbm.at[i2_v], r2_v, sem)
        g1.wait()
        g2.wait()

        def body(j, carry):
            wa = a_v[pl.ds(j, 16)][0]
            wb = b_v[pl.ds(j, 16)][0]
            row1 = r1_v.at[j]
            row2 = r2_v.at[j]
            for k0 in range(HID // 16):
                sl = pl.ds(k0 * 16, 16)
                row1[sl] = wa * row1[sl] + wb * row2[sl]
            return carry

        lax.fori_loop(0, TPW, body, 0)
        pltpu.sync_copy(r1_v, out_hbm.at[pl.ds(base, TPW)])

    return comb(og, p1f, p2f, w1f, w2f)


def _expert_grid_spec():
    return pltpu.PrefetchScalarGridSpec(
        num_scalar_prefetch=2,
        grid=(NT,),
        in_specs=[
            pl.BlockSpec((BLK, HID), lambda t, te, nt: (t, 0)),
            pl.BlockSpec((NE, HID, HID), lambda t, te, nt: (0, 0, 0)),
            pl.BlockSpec((NE, HID, HID), lambda t, te, nt: (0, 0, 0)),
            pl.BlockSpec((NE, 1, HID), lambda t, te, nt: (0, 0, 0)),
            pl.BlockSpec((NE, 1, HID), lambda t, te, nt: (0, 0, 0)),
            pl.BlockSpec((NE, HID, HID), lambda t, te, nt: (0, 0, 0)),
            pl.BlockSpec((NE, 1, HID), lambda t, te, nt: (0, 0, 0)),
        ],
        out_specs=pl.BlockSpec((BLK, HID), lambda t, te, nt: (t, 0)),
    )


def kernel(hidden_states, gate_w, mlp1_weight, mlp1_bias, mlp2_weight,
           mlp2_bias):
    h = hidden_states.reshape(TOK, HID)
    p1, p2, w1, w2, te, nt, aux = pl.pallas_call(
        _router_body,
        out_shape=[
            jax.ShapeDtypeStruct((TOK, 1), jnp.int32),
            jax.ShapeDtypeStruct((TOK, 1), jnp.int32),
            jax.ShapeDtypeStruct((TOK, 1), jnp.float32),
            jax.ShapeDtypeStruct((TOK, 1), jnp.float32),
            jax.ShapeDtypeStruct((NT, 1), jnp.int32),
            jax.ShapeDtypeStruct((1, 1), jnp.int32),
            jax.ShapeDtypeStruct((1, 1), jnp.float32),
        ],
    )(h, gate_w)
    p1f = p1.reshape(TOK)
    p2f = p2.reshape(TOK)
    xg = _dispatch(h, p1f, p2f)
    wg = lax.slice(mlp1_weight, (0, 0, 0), (NE, 2 * HID, HID),
                   (1, 2, 1)).astype(jnp.bfloat16)
    wl = lax.slice(mlp1_weight, (0, 1, 0), (NE, 2 * HID, HID),
                   (1, 2, 1)).astype(jnp.bfloat16)
    b1r = mlp1_bias.reshape(NE, 1, HID, 2)
    og = pl.pallas_call(
        _expert_body,
        grid_spec=_expert_grid_spec(),
        out_shape=jax.ShapeDtypeStruct((PAD_T, HID), jnp.float32),
        compiler_params=pltpu.CompilerParams(
            vmem_limit_bytes=60 * 1024 * 1024),
    )(te.reshape(NT), nt.reshape(1), xg, wg, wl,
      b1r[..., 0], b1r[..., 1], mlp2_weight.astype(jnp.bfloat16),
      mlp2_bias.reshape(NE, 1, HID))
    out = _combine(og, p1f, p2f, w1.reshape(TOK), w2.reshape(TOK))
    return out.reshape(1, TOK, HID), aux[0, 0]


# speedup vs baseline: 1.9730x; 1.9730x over previous
import functools

import jax
import jax.numpy as jnp
from jax import lax
from jax.experimental import pallas as pl
from jax.experimental.pallas import tpu as pltpu
from jax.experimental.pallas import tpu_sc as plsc

TOK = 2048
HID = 768
NE = 8
BLK = 256
NT = 24
PAD_T = NT * BLK
NW = 32
TPW = TOK // NW


def _router_body(h_ref, gw_ref, p1_ref, p2_ref, w1_ref, w2_ref, te_ref,
                 nt_ref, aux_ref):
    h = h_ref[...]
    gw = gw_ref[...]
    logits = lax.dot_general(h, gw, (((1,), (1,)), ((), ())),
                             preferred_element_type=jnp.float32)
    iota_e = lax.broadcasted_iota(jnp.int32, (TOK, NE), 1)
    m1 = jnp.max(logits, axis=1, keepdims=True)
    i1 = jnp.min(jnp.where(logits == m1, iota_e, NE), axis=1, keepdims=True)
    masked = jnp.where(iota_e == i1, -1e30, logits)
    m2 = jnp.max(masked, axis=1, keepdims=True)
    i2 = jnp.min(jnp.where(masked == m2, iota_e, NE), axis=1, keepdims=True)
    e2 = jnp.exp(m2 - m1)
    w1_ref[...] = (1.0 / (1.0 + e2)).reshape(TOK // 128, 128)
    w2_ref[...] = (e2 / (1.0 + e2)).reshape(TOK // 128, 128)
    p = jnp.exp(logits - m1)
    p = p / jnp.sum(p, axis=1, keepdims=True)
    avg_prob = jnp.mean(p, axis=0, keepdims=True)
    m1h = (iota_e == i1).astype(jnp.float32)
    m2h = (iota_e == i2).astype(jnp.float32)
    cnt1 = jnp.sum(m1h, axis=0, keepdims=True)
    cnt = cnt1 + jnp.sum(m2h, axis=0, keepdims=True)
    aux_ref[...] = jnp.sum(cnt * (1.0 / TOK) * avg_prob * (0.01 * NE),
                           axis=1, keepdims=True)
    rr = lax.broadcasted_iota(jnp.int32, (TOK, TOK), 0)
    cc = lax.broadcasted_iota(jnp.int32, (TOK, TOK), 1)
    tri = (cc < rr).astype(jnp.bfloat16)
    m12 = jnp.concatenate([m1h, m2h], axis=1).astype(jnp.bfloat16)
    csum = lax.dot_general(tri, m12, (((1,), (0,)), ((), ())),
                           preferred_element_type=jnp.float32)
    c1 = csum[:, :NE]
    c2 = csum[:, NE:]
    sz = jnp.floor((cnt + (BLK - 1.0)) * (1.0 / BLK)) * BLK
    re8 = lax.broadcasted_iota(jnp.int32, (NE, NE), 0)
    ce8 = lax.broadcasted_iota(jnp.int32, (NE, NE), 1)
    triu = (re8 < ce8).astype(jnp.float32)
    g = lax.dot_general(sz, triu, (((1,), (0,)), ((), ())),
                        preferred_element_type=jnp.float32)
    gnext = g + sz
    nrows = jnp.sum(sz, axis=1, keepdims=True)
    nt_ref[...] = (nrows * (1.0 / BLK)).astype(jnp.int32)
    p1 = jnp.sum(m1h * (g + c1), axis=1, keepdims=True)
    p2 = jnp.sum(m2h * (g + cnt1 + c2), axis=1, keepdims=True)
    p1_ref[...] = p1.astype(jnp.int32).reshape(TOK // 128, 128)
    p2_ref[...] = p2.astype(jnp.int32).reshape(TOK // 128, 128)
    tt = lax.broadcasted_iota(jnp.int32, (NT, NE), 0).astype(jnp.float32) \
        * float(BLK)
    texp = jnp.sum((gnext <= tt).astype(jnp.float32), axis=1, keepdims=True)
    maxte = jnp.sum((gnext <= nrows - BLK).astype(jnp.float32))
    te_ref[...] = jnp.minimum(texp, maxte).astype(jnp.int32)


def _expert_body(te_ref, nt_ref, x_ref, wg_ref, wl_ref, bg_ref, bl_ref,
                 w2_ref, b2_ref, o_ref):
    t = pl.program_id(0)

    @pl.when(t < nt_ref[0])
    def _():
        e = te_ref[t]
        x = x_ref[...].astype(jnp.bfloat16)
        hg = lax.dot_general(x, wg_ref[e], (((1,), (1,)), ((), ())),
                             preferred_element_type=jnp.float32)
        hl = lax.dot_general(x, wl_ref[e], (((1,), (1,)), ((), ())),
                             preferred_element_type=jnp.float32)
        xg = jnp.minimum(hg + bg_ref[e], 7.0)
        xl = jnp.clip(hl + bl_ref[e], -7.0, 7.0)
        act = xg * (1.0 / (1.0 + jnp.exp(-1.702 * xg))) * (xl + 1.0)
        o = lax.dot_general(act.astype(jnp.bfloat16), w2_ref[e],
                            (((1,), (1,)), ((), ())),
                            preferred_element_type=jnp.float32)
        o_ref[...] = o + b2_ref[e]


def _sc_mesh():
    return plsc.VectorSubcoreMesh(core_axis_name="c", subcore_axis_name="s")


def _dispatch(h, p1f, p2f):
    @functools.partial(
        pl.kernel,
        out_type=jax.ShapeDtypeStruct((PAD_T, HID), jnp.float32),
        mesh=_sc_mesh(),
        scratch_types=[
            pltpu.VMEM((TPW,), jnp.int32),
            pltpu.VMEM((TPW,), jnp.int32),
            pltpu.VMEM((TPW, HID), jnp.float32),
            pltpu.SemaphoreType.DMA,
            pltpu.SemaphoreType.DMA,
        ],
    )
    def disp(h_hbm, p1_hbm, p2_hbm, xg_hbm, i1_v, i2_v, rows_v, sem, sem2):
        wid = lax.axis_index("s") * 2 + lax.axis_index("c")
        base = wid * TPW
        a1 = pltpu.async_copy(p1_hbm.at[pl.ds(base, TPW)], i1_v, sem2)
        a2 = pltpu.async_copy(p2_hbm.at[pl.ds(base, TPW)], i2_v, sem2)
        a3 = pltpu.async_copy(h_hbm.at[pl.ds(base, TPW)], rows_v, sem2)
        a1.wait()
        a2.wait()
        a3.wait()
        c1 = pltpu.async_copy(rows_v, xg_hbm.at[i1_v], sem)
        c2 = pltpu.async_copy(rows_v, xg_hbm.at[i2_v], sem)
        c1.wait()
        c2.wait()

    return disp(h, p1f, p2f)


def _combine(og, p1f, p2f, w1f, w2f):
    @functools.partial(
        pl.kernel,
        out_type=jax.ShapeDtypeStruct((TOK, HID), jnp.float32),
        mesh=_sc_mesh(),
        scratch_types=[
            pltpu.VMEM((TPW,), jnp.int32),
            pltpu.VMEM((TPW,), jnp.int32),
            pltpu.VMEM((TPW + 16,), jnp.float32),
            pltpu.VMEM((TPW + 16,), jnp.float32),
            pltpu.VMEM((TPW, HID), jnp.float32),
            pltpu.VMEM((TPW, HID), jnp.float32),
            pltpu.SemaphoreType.DMA,
            pltpu.SemaphoreType.DMA,
        ],
    )
    def comb(og_hbm, p1_hbm, p2_hbm, w1_hbm, w2_hbm, out_hbm,
             i1_v, i2_v, a_v, b_v, r1_v, r2_v, sem, sem2):
        wid = lax.axis_index("s") * 2 + lax.axis_index("c")
        base = wid * TPW
        a1 = pltpu.async_copy(p1_hbm.at[pl.ds(base, TPW)], i1_v, sem2)
        a2 = pltpu.async_copy(p2_hbm.at[pl.ds(base, TPW)], i2_v, sem2)
        a3 = pltpu.async_copy(w1_hbm.at[pl.ds(base, TPW)],
                              a_v.at[pl.ds(0, TPW)], sem2)
        a4 = pltpu.async_copy(w2_hbm.at[pl.ds(base, TPW)],
                              b_v.at[pl.ds(0, TPW)], sem2)
        a1.wait()
        a2.wait()
        a3.wait()
        a4.wait()
        g1 = pltpu.async_copy(og_hbm.at[i1_v], r1_v, sem)
        g2 = pltpu.async_copy(og_hbm.at[i2_v], r2_v, sem)
        g1.wait()
        g2.wait()

        def body(j, carry):
            wa = a_v[pl.ds(j, 16)][0]
            wb = b_v[pl.ds(j, 16)][0]
            row1 = r1_v.at[j]
            row2 = r2_v.at[j]
            for k0 in range(HID // 16):
                sl = pl.ds(k0 * 16, 16)
                row1[sl] = wa * row1[sl] + wb * row2[sl]
            return carry

        lax.fori_loop(0, TPW, body, 0)
        pltpu.sync_copy(r1_v, out_hbm.at[pl.ds(base, TPW)])

    return comb(og, p1f, p2f, w1f, w2f)


def _expert_grid_spec():
    return pltpu.PrefetchScalarGridSpec(
        num_scalar_prefetch=2,
        grid=(NT,),
        in_specs=[
            pl.BlockSpec((BLK, HID), lambda t, te, nt: (t, 0)),
            pl.BlockSpec((NE, HID, HID), lambda t, te, nt: (0, 0, 0)),
            pl.BlockSpec((NE, HID, HID), lambda t, te, nt: (0, 0, 0)),
            pl.BlockSpec((NE, 1, HID), lambda t, te, nt: (0, 0, 0)),
            pl.BlockSpec((NE, 1, HID), lambda t, te, nt: (0, 0, 0)),
            pl.BlockSpec((NE, HID, HID), lambda t, te, nt: (0, 0, 0)),
            pl.BlockSpec((NE, 1, HID), lambda t, te, nt: (0, 0, 0)),
        ],
        out_specs=pl.BlockSpec((BLK, HID), lambda t, te, nt: (t, 0)),
    )


def kernel(hidden_states, gate_w, mlp1_weight, mlp1_bias, mlp2_weight,
           mlp2_bias):
    h = hidden_states.reshape(TOK, HID)
    p1, p2, w1, w2, te, nt, aux = pl.pallas_call(
        _router_body,
        out_shape=[
            jax.ShapeDtypeStruct((TOK // 128, 128), jnp.int32),
            jax.ShapeDtypeStruct((TOK // 128, 128), jnp.int32),
            jax.ShapeDtypeStruct((TOK // 128, 128), jnp.float32),
            jax.ShapeDtypeStruct((TOK // 128, 128), jnp.float32),
            jax.ShapeDtypeStruct((NT, 1), jnp.int32),
            jax.ShapeDtypeStruct((1, 1), jnp.int32),
            jax.ShapeDtypeStruct((1, 1), jnp.float32),
        ],
    )(h, gate_w)
    p1f = p1.reshape(TOK)
    p2f = p2.reshape(TOK)
    xg = _dispatch(h, p1f, p2f)
    w1b = mlp1_weight.astype(jnp.bfloat16).reshape(NE, HID, 2, HID)
    wg = w1b[:, :, 0, :]
    wl = w1b[:, :, 1, :]
    b1r = mlp1_bias.reshape(NE, 1, HID, 2)
    og = pl.pallas_call(
        _expert_body,
        grid_spec=_expert_grid_spec(),
        out_shape=jax.ShapeDtypeStruct((PAD_T, HID), jnp.float32),
        compiler_params=pltpu.CompilerParams(
            vmem_limit_bytes=60 * 1024 * 1024),
    )(te.reshape(NT), nt.reshape(1), xg, wg, wl,
      b1r[..., 0], b1r[..., 1], mlp2_weight.astype(jnp.bfloat16),
      mlp2_bias.reshape(NE, 1, HID))
    out = _combine(og, p1f, p2f, w1.reshape(TOK), w2.reshape(TOK))
    return out.reshape(1, TOK, HID), aux[0, 0]


# speedup vs baseline: 1.9922x; 1.0098x over previous
import functools

import jax
import jax.numpy as jnp
from jax import lax
from jax.experimental import pallas as pl
from jax.experimental.pallas import tpu as pltpu
from jax.experimental.pallas import tpu_sc as plsc

TOK = 2048
HID = 768
NE = 8
BLK = 256
NT = 24
PAD_T = NT * BLK
NW = 32
TPW = TOK // NW


def _router_body(h_ref, gw_ref, p1_ref, p2_ref, w1_ref, w2_ref, te_ref,
                 nt_ref, aux_ref):
    h = h_ref[...]
    gw = gw_ref[...]
    logits = lax.dot_general(h, gw, (((1,), (1,)), ((), ())),
                             preferred_element_type=jnp.float32)
    iota_e = lax.broadcasted_iota(jnp.int32, (TOK, NE), 1)
    m1 = jnp.max(logits, axis=1, keepdims=True)
    i1 = jnp.min(jnp.where(logits == m1, iota_e, NE), axis=1, keepdims=True)
    masked = jnp.where(iota_e == i1, -1e30, logits)
    m2 = jnp.max(masked, axis=1, keepdims=True)
    i2 = jnp.min(jnp.where(masked == m2, iota_e, NE), axis=1, keepdims=True)
    e2 = jnp.exp(m2 - m1)
    w1_ref[...] = 1.0 / (1.0 + e2)
    w2_ref[...] = e2 / (1.0 + e2)
    p = jnp.exp(logits - m1)
    p = p / jnp.sum(p, axis=1, keepdims=True)
    avg_prob = jnp.mean(p, axis=0, keepdims=True)
    m1h = (iota_e == i1).astype(jnp.float32)
    m2h = (iota_e == i2).astype(jnp.float32)
    cnt1 = jnp.sum(m1h, axis=0, keepdims=True)
    cnt = cnt1 + jnp.sum(m2h, axis=0, keepdims=True)
    aux_ref[...] = jnp.sum(cnt * (1.0 / TOK) * avg_prob * (0.01 * NE),
                           axis=1, keepdims=True)
    rr = lax.broadcasted_iota(jnp.int32, (TOK, TOK), 0)
    cc = lax.broadcasted_iota(jnp.int32, (TOK, TOK), 1)
    tri = (cc < rr).astype(jnp.bfloat16)
    m12 = jnp.concatenate([m1h, m2h], axis=1).astype(jnp.bfloat16)
    csum = lax.dot_general(tri, m12, (((1,), (0,)), ((), ())),
                           preferred_element_type=jnp.float32)
    c1 = csum[:, :NE]
    c2 = csum[:, NE:]
    sz = jnp.floor((cnt + (BLK - 1.0)) * (1.0 / BLK)) * BLK
    re8 = lax.broadcasted_iota(jnp.int32, (NE, NE), 0)
    ce8 = lax.broadcasted_iota(jnp.int32, (NE, NE), 1)
    triu = (re8 < ce8).astype(jnp.float32)
    g = lax.dot_general(sz, triu, (((1,), (0,)), ((), ())),
                        preferred_element_type=jnp.float32)
    gnext = g + sz
    nrows = jnp.sum(sz, axis=1, keepdims=True)
    nt_ref[...] = (nrows * (1.0 / BLK)).astype(jnp.int32)
    p1 = jnp.sum(m1h * (g + c1), axis=1, keepdims=True)
    p2 = jnp.sum(m2h * (g + cnt1 + c2), axis=1, keepdims=True)
    p1_ref[...] = p1.astype(jnp.int32)
    p2_ref[...] = p2.astype(jnp.int32)
    tt = lax.broadcasted_iota(jnp.int32, (NT, NE), 0).astype(jnp.float32) \
        * float(BLK)
    texp = jnp.sum((gnext <= tt).astype(jnp.float32), axis=1, keepdims=True)
    maxte = jnp.sum((gnext <= nrows - BLK).astype(jnp.float32))
    te_ref[...] = jnp.minimum(texp, maxte).astype(jnp.int32)


def _expert_body(te_ref, nt_ref, x_ref, wg_ref, wl_ref, bg_ref, bl_ref,
                 w2_ref, b2_ref, o_ref):
    t = pl.program_id(0)

    @pl.when(t < nt_ref[0])
    def _():
        e = te_ref[t]
        x = x_ref[...].astype(jnp.bfloat16)
        hg = lax.dot_general(x, wg_ref[e], (((1,), (1,)), ((), ())),
                             preferred_element_type=jnp.float32)
        hl = lax.dot_general(x, wl_ref[e], (((1,), (1,)), ((), ())),
                             preferred_element_type=jnp.float32)
        xg = jnp.minimum(hg + bg_ref[e], 7.0)
        xl = jnp.clip(hl + bl_ref[e], -7.0, 7.0)
        act = xg * (1.0 / (1.0 + jnp.exp(-1.702 * xg))) * (xl + 1.0)
        o = lax.dot_general(act.astype(jnp.bfloat16), w2_ref[e],
                            (((1,), (1,)), ((), ())),
                            preferred_element_type=jnp.float32)
        o_ref[...] = o + b2_ref[e]


def _sc_mesh():
    return plsc.VectorSubcoreMesh(core_axis_name="c", subcore_axis_name="s")


def _dispatch(h, p1f, p2f):
    @functools.partial(
        pl.kernel,
        out_type=jax.ShapeDtypeStruct((PAD_T, HID), jnp.float32),
        mesh=_sc_mesh(),
        scratch_types=[
            pltpu.VMEM((TPW,), jnp.int32),
            pltpu.VMEM((TPW,), jnp.int32),
            pltpu.VMEM((TPW, HID), jnp.float32),
            pltpu.SemaphoreType.DMA,
            pltpu.SemaphoreType.DMA,
        ],
    )
    def disp(h_hbm, p1_hbm, p2_hbm, xg_hbm, i1_v, i2_v, rows_v, sem, sem2):
        wid = lax.axis_index("s") * 2 + lax.axis_index("c")
        base = wid * TPW
        a1 = pltpu.async_copy(p1_hbm.at[pl.ds(base, TPW)], i1_v, sem2)
        a2 = pltpu.async_copy(p2_hbm.at[pl.ds(base, TPW)], i2_v, sem2)
        a3 = pltpu.async_copy(h_hbm.at[pl.ds(base, TPW)], rows_v, sem2)
        a1.wait()
        a2.wait()
        a3.wait()
        c1 = pltpu.async_copy(rows_v, xg_hbm.at[i1_v], sem)
        c2 = pltpu.async_copy(rows_v, xg_hbm.at[i2_v], sem)
        c1.wait()
        c2.wait()

    return disp(h, p1f, p2f)


def _combine(og, p1f, p2f, w1f, w2f):
    @functools.partial(
        pl.kernel,
        out_type=jax.ShapeDtypeStruct((TOK, HID), jnp.float32),
        mesh=_sc_mesh(),
        scratch_types=[
            pltpu.VMEM((TPW,), jnp.int32),
            pltpu.VMEM((TPW,), jnp.int32),
            pltpu.VMEM((TPW + 16,), jnp.float32),
            pltpu.VMEM((TPW + 16,), jnp.float32),
            pltpu.VMEM((TPW, HID), jnp.float32),
            pltpu.VMEM((TPW, HID), jnp.float32),
            pltpu.SemaphoreType.DMA,
            pltpu.SemaphoreType.DMA,
        ],
    )
    def comb(og_hbm, p1_hbm, p2_hbm, w1_hbm, w2_hbm, out_hbm,
             i1_v, i2_v, a_v, b_v, r1_v, r2_v, sem, sem2):
        wid = lax.axis_index("s") * 2 + lax.axis_index("c")
        base = wid * TPW
        a1 = pltpu.async_copy(p1_hbm.at[pl.ds(base, TPW)], i1_v, sem2)
        a2 = pltpu.async_copy(p2_hbm.at[pl.ds(base, TPW)], i2_v, sem2)
        a3 = pltpu.async_copy(w1_hbm.at[pl.ds(base, TPW)],
                              a_v.at[pl.ds(0, TPW)], sem2)
        a4 = pltpu.async_copy(w2_hbm.at[pl.ds(base, TPW)],
                              b_v.at[pl.ds(0, TPW)], sem2)
        a1.wait()
        a2.wait()
        a3.wait()
        a4.wait()
        g1 = pltpu.async_copy(og_hbm.at[i1_v], r1_v, sem)
        g2 = pltpu.async_copy(og_hbm.at[i2_v], r2_v, sem)
        g1.wait()
        g2.wait()

        def body(j, carry):
            wa = a_v[pl.ds(j, 16)][0]
            wb = b_v[pl.ds(j, 16)][0]
            row1 = r1_v.at[j]
            row2 = r2_v.at[j]
            for k0 in range(HID // 16):
                sl = pl.ds(k0 * 16, 16)
                row1[sl] = wa * row1[sl] + wb * row2[sl]
            return carry

        lax.fori_loop(0, TPW, body, 0)
        pltpu.sync_copy(r1_v, out_hbm.at[pl.ds(base, TPW)])

    return comb(og, p1f, p2f, w1f, w2f)


def _expert_grid_spec():
    return pltpu.PrefetchScalarGridSpec(
        num_scalar_prefetch=2,
        grid=(NT,),
        in_specs=[
            pl.BlockSpec((BLK, HID), lambda t, te, nt: (t, 0)),
            pl.BlockSpec((NE, HID, HID), lambda t, te, nt: (0, 0, 0)),
            pl.BlockSpec((NE, HID, HID), lambda t, te, nt: (0, 0, 0)),
            pl.BlockSpec((NE, 1, HID), lambda t, te, nt: (0, 0, 0)),
            pl.BlockSpec((NE, 1, HID), lambda t, te, nt: (0, 0, 0)),
            pl.BlockSpec((NE, HID, HID), lambda t, te, nt: (0, 0, 0)),
            pl.BlockSpec((NE, 1, HID), lambda t, te, nt: (0, 0, 0)),
        ],
        out_specs=pl.BlockSpec((BLK, HID), lambda t, te, nt: (t, 0)),
    )


def kernel(hidden_states, gate_w, mlp1_weight, mlp1_bias, mlp2_weight,
           mlp2_bias):
    h = hidden_states.reshape(TOK, HID)
    p1, p2, w1, w2, te, nt, aux = pl.pallas_call(
        _router_body,
        out_shape=[
            jax.ShapeDtypeStruct((TOK, 1), jnp.int32),
            jax.ShapeDtypeStruct((TOK, 1), jnp.int32),
            jax.ShapeDtypeStruct((TOK, 1), jnp.float32),
            jax.ShapeDtypeStruct((TOK, 1), jnp.float32),
            jax.ShapeDtypeStruct((NT, 1), jnp.int32),
            jax.ShapeDtypeStruct((1, 1), jnp.int32),
            jax.ShapeDtypeStruct((1, 1), jnp.float32),
        ],
    )(h, gate_w)
    p1f = p1.reshape(TOK)
    p2f = p2.reshape(TOK)
    xg = _dispatch(h, p1f, p2f)
    w1b = mlp1_weight.astype(jnp.bfloat16).reshape(NE, HID, 2, HID)
    wg = w1b[:, :, 0, :]
    wl = w1b[:, :, 1, :]
    b1r = mlp1_bias.reshape(NE, 1, HID, 2)
    og = pl.pallas_call(
        _expert_body,
        grid_spec=_expert_grid_spec(),
        out_shape=jax.ShapeDtypeStruct((PAD_T, HID), jnp.float32),
        compiler_params=pltpu.CompilerParams(
            vmem_limit_bytes=60 * 1024 * 1024),
    )(te.reshape(NT), nt.reshape(1), xg, wg, wl,
      b1r[..., 0], b1r[..., 1], mlp2_weight.astype(jnp.bfloat16),
      mlp2_bias.reshape(NE, 1, HID))
    out = _combine(og, p1f, p2f, w1.reshape(TOK), w2.reshape(TOK))
    return out.reshape(1, TOK, HID), aux[0, 0]
